# Initial kernel scaffold; baseline (speedup 1.0000x reference)
import functools

import jax
import jax.numpy as jnp
from jax import lax
from jax.experimental import pallas as pl
from jax.experimental.pallas import tpu as pltpu
from jax.experimental.pallas import tpu_sc as plsc

N_NODES = 10000
N_EDGES = 320000
D_IN = 128
D_HID = 256
D_OUT = 64

NC = 2
NS = 16
CHUNK = 128
NCHUNKS = N_EDGES // CHUNK
T_STEPS = -(-NCHUNKS // (NC * NS))
ROWS_PER_TILE = N_NODES // NS
ZROWS = 125


def _make_sc_spmm(d, gather):
    mesh = plsc.VectorSubcoreMesh(
        core_axis_name="c", subcore_axis_name="s",
        num_cores=NC, num_subcores=NS)
    scratch = [
        pltpu.VMEM((CHUNK,), jnp.int32),
        pltpu.VMEM((CHUNK,), jnp.int32),
        pltpu.VMEM((CHUNK, d), jnp.float32),
        pltpu.VMEM((ZROWS, d), jnp.float32),
        pltpu.VMEM_SHARED((N_NODES, d), jnp.float32),
        pltpu.SemaphoreType.DMA,
    ]
    out_type = jax.ShapeDtypeStruct((NC, N_NODES, d), jnp.float32)

    def body(table_hbm, src_hbm, dst_hbm, out_hbm,
             srcidx, dstidx, rows, zbuf, acc, sem):
        cid = lax.axis_index("c")
        sid = lax.axis_index("s")
        wid = cid * NS + sid

        def fill(ref, nrows, value):
            def outer(i, carry):
                def inner(j, carry2):
                    ref[i, pl.ds(j * 16, 16)] = jnp.full(
                        (16,), value, jnp.float32)
                    return carry2
                return lax.fori_loop(0, d // 16, inner, carry)
            lax.fori_loop(0, nrows, outer, 0)

        fill(zbuf, ZROWS, 0.0)
        if not gather:
            fill(rows, CHUNK, 1.0)

        r0 = sid * ROWS_PER_TILE
        for j in range(ROWS_PER_TILE // ZROWS):
            pltpu.sync_copy(zbuf, acc.at[pl.ds(r0 + j * ZROWS, ZROWS)])
        plsc.subcore_barrier()

        def step(t, carry):
            chunk = wid + NC * NS * t

            @pl.when(chunk < NCHUNKS)
            def _():
                base = chunk * CHUNK
                pltpu.sync_copy(dst_hbm.at[pl.ds(base, CHUNK)], dstidx)
                if gather:
                    pltpu.sync_copy(src_hbm.at[pl.ds(base, CHUNK)], srcidx)
                    pltpu.async_copy(table_hbm.at[srcidx], rows, sem).wait()
                pltpu.sync_copy(rows, acc.at[dstidx], add=True)
            return carry

        lax.fori_loop(0, T_STEPS, step, 0)
        plsc.subcore_barrier()
        pltpu.sync_copy(acc.at[pl.ds(r0, ROWS_PER_TILE)],
                        out_hbm.at[cid, pl.ds(r0, ROWS_PER_TILE)])

    if gather:
        kern = body
    else:
        def kern(src_hbm, dst_hbm, out_hbm, *rest):
            return body(None, src_hbm, dst_hbm, out_hbm, *rest)

    return functools.partial(
        pl.kernel, out_type=out_type, mesh=mesh, scratch_types=scratch,
        name=f"sc_spmm_{d}_{int(gather)}")(kern)


_sc_deg = _make_sc_spmm(16, gather=False)
_sc_spmm128 = _make_sc_spmm(D_IN, gather=True)
_sc_spmm64 = _make_sc_spmm(D_OUT, gather=True)

_BLK = 1000
_GRID = N_NODES // _BLK


def _prescale_body(degp_ref, x_ref, dinv_ref, xs_ref):
    deg = degp_ref[0] + degp_ref[1] + 1.0
    dinv = lax.rsqrt(deg)[:, 0:1]
    dinv_ref[...] = dinv
    xs_ref[...] = dinv * x_ref[...]


_tc_prescale = pl.pallas_call(
    _prescale_body,
    grid=(_GRID,),
    in_specs=[
        pl.BlockSpec((NC, _BLK, 16), lambda i: (0, i, 0)),
        pl.BlockSpec((_BLK, D_IN), lambda i: (i, 0)),
    ],
    out_specs=[
        pl.BlockSpec((_BLK, 1), lambda i: (i, 0)),
        pl.BlockSpec((_BLK, D_IN), lambda i: (i, 0)),
    ],
    out_shape=[
        jax.ShapeDtypeStruct((N_NODES, 1), jnp.float32),
        jax.ShapeDtypeStruct((N_NODES, D_IN), jnp.float32),
    ],
)


def _mid_body(y1p_ref, xs_ref, dinv_ref, w1_ref, b1_ref, w2_ref, gs_ref):
    z = dinv_ref[...] * (y1p_ref[0] + y1p_ref[1] + xs_ref[...])
    h = jnp.dot(z, w1_ref[...], preferred_element_type=jnp.float32)
    h = jnp.maximum(h + b1_ref[...], 0.0)
    g = jnp.dot(h, w2_ref[...], preferred_element_type=jnp.float32)
    gs_ref[...] = dinv_ref[...] * g


_tc_mid = pl.pallas_call(
    _mid_body,
    grid=(_GRID,),
    in_specs=[
        pl.BlockSpec((NC, _BLK, D_IN), lambda i: (0, i, 0)),
        pl.BlockSpec((_BLK, D_IN), lambda i: (i, 0)),
        pl.BlockSpec((_BLK, 1), lambda i: (i, 0)),
        pl.BlockSpec((D_IN, D_HID), lambda i: (0, 0)),
        pl.BlockSpec((1, D_HID), lambda i: (0, 0)),
        pl.BlockSpec((D_HID, D_OUT), lambda i: (0, 0)),
    ],
    out_specs=pl.BlockSpec((_BLK, D_OUT), lambda i: (i, 0)),
    out_shape=jax.ShapeDtypeStruct((N_NODES, D_OUT), jnp.float32),
)


def _final_body(y2p_ref, gs_ref, dinv_ref, b2_ref, out_ref):
    t = dinv_ref[...] * (y2p_ref[0] + y2p_ref[1] + gs_ref[...]) + b2_ref[...]
    m = jnp.max(t, axis=1, keepdims=True)
    e = jnp.exp(t - m)
    s = jnp.sum(e, axis=1, keepdims=True)
    out_ref[...] = (t - m) - jnp.log(s)


_tc_final = pl.pallas_call(
    _final_body,
    grid=(_GRID,),
    in_specs=[
        pl.BlockSpec((NC, _BLK, D_OUT), lambda i: (0, i, 0)),
        pl.BlockSpec((_BLK, D_OUT), lambda i: (i, 0)),
        pl.BlockSpec((_BLK, 1), lambda i: (i, 0)),
        pl.BlockSpec((1, D_OUT), lambda i: (0, 0)),
    ],
    out_specs=pl.BlockSpec((_BLK, D_OUT), lambda i: (i, 0)),
    out_shape=jax.ShapeDtypeStruct((N_NODES, D_OUT), jnp.float32),
)


@jax.jit
def kernel(x, edge_index, W1, b1, W2, b2):
    src = edge_index[0]
    dst = edge_index[1]
    degp = _sc_deg(src, dst)
    dinv, xs = _tc_prescale(degp, x)
    y1p = _sc_spmm128(xs, src, dst)
    gs = _tc_mid(y1p, xs, dinv, W1, b1.reshape(1, D_HID), W2)
    y2p = _sc_spmm64(gs, src, dst)
    return _tc_final(y2p, gs, dinv, b2.reshape(1, D_OUT))


# speedup vs baseline: 19.3227x; 19.3227x over previous
import functools

import jax
import jax.numpy as jnp
from jax import lax
from jax.experimental import pallas as pl
from jax.experimental.pallas import tpu as pltpu
from jax.experimental.pallas import tpu_sc as plsc

N_NODES = 10000
N_PAD = 10240
N_EDGES = 320000
D_IN = 128
D_HID = 256
D_OUT = 64

NC = 2
NS = 16
CHUNK = 128
NCHUNKS = N_EDGES // CHUNK
T_STEPS = -(-NCHUNKS // (NC * NS))
ROWS_PER_TILE = N_PAD // NS
ZROWS = 128


def _make_sc_spmm(d, gather):
    mesh = plsc.VectorSubcoreMesh(
        core_axis_name="c", subcore_axis_name="s",
        num_cores=NC, num_subcores=NS)
    scratch = [
        pltpu.VMEM((CHUNK,), jnp.int32),
        pltpu.VMEM((CHUNK,), jnp.int32),
        pltpu.VMEM((CHUNK, d), jnp.float32),
        pltpu.VMEM((ZROWS, d), jnp.float32),
        pltpu.VMEM_SHARED((N_PAD, d), jnp.float32),
        pltpu.SemaphoreType.DMA,
    ]
    out_type = jax.ShapeDtypeStruct((NC, N_PAD, d), jnp.float32)

    def body(table_hbm, src_hbm, dst_hbm, out_hbm,
             srcidx, dstidx, rows, zbuf, acc, sem):
        cid = lax.axis_index("c")
        sid = lax.axis_index("s")
        wid = cid * NS + sid

        def fill(ref, nrows, value):
            def outer(i, carry):
                def inner(j, carry2):
                    ref[i, pl.ds(j * 16, 16)] = jnp.full(
                        (16,), value, jnp.float32)
                    return carry2
                return lax.fori_loop(0, d // 16, inner, carry)
            lax.fori_loop(0, nrows, outer, 0)

        fill(zbuf, ZROWS, 0.0)
        if not gather:
            fill(rows, CHUNK, 1.0)

        r0 = sid * ROWS_PER_TILE
        for j in range(ROWS_PER_TILE // ZROWS):
            pltpu.sync_copy(zbuf, acc.at[pl.ds(r0 + j * ZROWS, ZROWS)])
        plsc.subcore_barrier()

        def step(t, carry):
            chunk = wid + NC * NS * t

            @pl.when(chunk < NCHUNKS)
            def _():
                base = chunk * CHUNK
                pltpu.sync_copy(dst_hbm.at[pl.ds(base, CHUNK)], dstidx)
                if gather:
                    pltpu.sync_copy(src_hbm.at[pl.ds(base, CHUNK)], srcidx)
                    pltpu.async_copy(table_hbm.at[srcidx], rows, sem).wait()
                pltpu.sync_copy(rows, acc.at[dstidx], add=True)
            return carry

        lax.fori_loop(0, T_STEPS, step, 0)
        plsc.subcore_barrier()
        pltpu.sync_copy(acc.at[pl.ds(r0, ROWS_PER_TILE)],
                        out_hbm.at[cid, pl.ds(r0, ROWS_PER_TILE)])

    if gather:
        kern = body
    else:
        def kern(src_hbm, dst_hbm, out_hbm, *rest):
            return body(None, src_hbm, dst_hbm, out_hbm, *rest)

    return functools.partial(
        pl.kernel, out_type=out_type, mesh=mesh, scratch_types=scratch,
        compiler_params=pltpu.CompilerParams(use_tc_tiling_on_sc=False),
        name=f"sc_spmm_{d}_{int(gather)}")(kern)


_sc_deg = _make_sc_spmm(16, gather=False)
_sc_spmm128 = _make_sc_spmm(D_IN, gather=True)
_sc_spmm64 = _make_sc_spmm(D_OUT, gather=True)

_BLK = 1000
_GRID = N_NODES // _BLK


def _prescale_body(degp_ref, x_ref, dinv_ref, xs_ref):
    deg = degp_ref[0] + degp_ref[1] + 1.0
    dinv = lax.rsqrt(deg)[:, 0:1]
    dinv_ref[...] = dinv
    xs_ref[...] = dinv * x_ref[...]


_tc_prescale = pl.pallas_call(
    _prescale_body,
    grid=(_GRID,),
    in_specs=[
        pl.BlockSpec((NC, _BLK, 16), lambda i: (0, i, 0)),
        pl.BlockSpec((_BLK, D_IN), lambda i: (i, 0)),
    ],
    out_specs=[
        pl.BlockSpec((_BLK, 1), lambda i: (i, 0)),
        pl.BlockSpec((_BLK, D_IN), lambda i: (i, 0)),
    ],
    out_shape=[
        jax.ShapeDtypeStruct((N_NODES, 1), jnp.float32),
        jax.ShapeDtypeStruct((N_NODES, D_IN), jnp.float32),
    ],
)


def _mid_body(y1p_ref, xs_ref, dinv_ref, w1_ref, b1_ref, w2_ref, gs_ref):
    z = dinv_ref[...] * (y1p_ref[0] + y1p_ref[1] + xs_ref[...])
    h = jnp.dot(z, w1_ref[...], preferred_element_type=jnp.float32)
    h = jnp.maximum(h + b1_ref[...], 0.0)
    g = jnp.dot(h, w2_ref[...], preferred_element_type=jnp.float32)
    gs_ref[...] = dinv_ref[...] * g


_tc_mid = pl.pallas_call(
    _mid_body,
    grid=(_GRID,),
    in_specs=[
        pl.BlockSpec((NC, _BLK, D_IN), lambda i: (0, i, 0)),
        pl.BlockSpec((_BLK, D_IN), lambda i: (i, 0)),
        pl.BlockSpec((_BLK, 1), lambda i: (i, 0)),
        pl.BlockSpec((D_IN, D_HID), lambda i: (0, 0)),
        pl.BlockSpec((1, D_HID), lambda i: (0, 0)),
        pl.BlockSpec((D_HID, D_OUT), lambda i: (0, 0)),
    ],
    out_specs=pl.BlockSpec((_BLK, D_OUT), lambda i: (i, 0)),
    out_shape=jax.ShapeDtypeStruct((N_NODES, D_OUT), jnp.float32),
)


def _final_body(y2p_ref, gs_ref, dinv_ref, b2_ref, out_ref):
    t = dinv_ref[...] * (y2p_ref[0] + y2p_ref[1] + gs_ref[...]) + b2_ref[...]
    m = jnp.max(t, axis=1, keepdims=True)
    e = jnp.exp(t - m)
    s = jnp.sum(e, axis=1, keepdims=True)
    out_ref[...] = (t - m) - jnp.log(s)


_tc_final = pl.pallas_call(
    _final_body,
    grid=(_GRID,),
    in_specs=[
        pl.BlockSpec((NC, _BLK, D_OUT), lambda i: (0, i, 0)),
        pl.BlockSpec((_BLK, D_OUT), lambda i: (i, 0)),
        pl.BlockSpec((_BLK, 1), lambda i: (i, 0)),
        pl.BlockSpec((1, D_OUT), lambda i: (0, 0)),
    ],
    out_specs=pl.BlockSpec((_BLK, D_OUT), lambda i: (i, 0)),
    out_shape=jax.ShapeDtypeStruct((N_NODES, D_OUT), jnp.float32),
)


@jax.jit
def kernel(x, edge_index, W1, b1, W2, b2):
    src = edge_index[0]
    dst = edge_index[1]
    degp = _sc_deg(src, dst)
    dinv, xs = _tc_prescale(degp, x)
    y1p = _sc_spmm128(xs, src, dst)
    gs = _tc_mid(y1p, xs, dinv, W1, b1.reshape(1, D_HID), W2)
    y2p = _sc_spmm64(gs, src, dst)
    return _tc_final(y2p, gs, dinv, b2.reshape(1, D_OUT))


# speedup vs baseline: 35.4376x; 1.8340x over previous
import functools

import jax
import jax.numpy as jnp
from jax import lax
from jax.experimental import pallas as pl
from jax.experimental.pallas import tpu as pltpu
from jax.experimental.pallas import tpu_sc as plsc

N_NODES = 10000
N_PAD = 10240
N_EDGES = 320000
D_IN = 128
D_HID = 256
D_OUT = 64

NC = 2
NS = 16
CHUNK = 128
NCHUNKS = N_EDGES // CHUNK
T_STEPS = -(-NCHUNKS // (NC * NS))
ROWS_PER_TILE = N_PAD // NS
ZROWS = 128


NBUF = 2
ISLOTS = 4


def _make_sc_spmm(d, gather):
    mesh = plsc.VectorSubcoreMesh(
        core_axis_name="c", subcore_axis_name="s",
        num_cores=NC, num_subcores=NS)
    if gather:
        scratch = [
            [pltpu.VMEM((CHUNK,), jnp.int32) for _ in range(ISLOTS)],
            [pltpu.VMEM((CHUNK,), jnp.int32) for _ in range(ISLOTS)],
            [pltpu.VMEM((CHUNK, d), jnp.float32) for _ in range(NBUF)],
            pltpu.VMEM_SHARED((N_PAD, d), jnp.float32),
            [pltpu.SemaphoreType.DMA for _ in range(ISLOTS)],
            [pltpu.SemaphoreType.DMA for _ in range(NBUF)],
            [pltpu.SemaphoreType.DMA for _ in range(NBUF)],
        ]
    else:
        scratch = [
            pltpu.VMEM((T_STEPS, CHUNK), jnp.int32),
            pltpu.VMEM((CHUNK, d), jnp.float32),
            pltpu.VMEM((ZROWS, d), jnp.float32),
            pltpu.VMEM_SHARED((N_PAD, d), jnp.float32),
            pltpu.SemaphoreType.DMA,
        ]
    out_type = jax.ShapeDtypeStruct((NC, N_PAD, d), jnp.float32)

    def fill(ref, nrows, value):
        def outer(i, carry):
            def inner(j, carry2):
                ref[i, pl.ds(j * 16, 16)] = jnp.full((16,), value,
                                                     jnp.float32)
                return carry2
            return lax.fori_loop(0, d // 16, inner, carry)
        lax.fori_loop(0, nrows, outer, 0)

    def worker_range():
        cid = lax.axis_index("c")
        sid = lax.axis_index("s")
        wid = cid * NS + sid
        start = wid * 78 + jnp.minimum(wid, 4)
        nch = jnp.where(wid < 4, 79, 78)
        return cid, sid, start, nch

    def gather_body(table_hbm, src_hbm, dst_hbm, out_hbm,
                    srcs, dsts, rows, acc, isems, gsems, ssems):
        cid, sid, start, nch = worker_range()

        fill(rows[0], CHUNK, 0.0)
        r0 = sid * ROWS_PER_TILE
        for j in range(ROWS_PER_TILE // CHUNK):
            pltpu.sync_copy(rows[0], acc.at[pl.ds(r0 + j * CHUNK, CHUNK)])
        plsc.subcore_barrier()

        def idx_load(t, slot):
            pltpu.async_copy(src_hbm.at[start + t], srcs[slot], isems[slot])
            pltpu.async_copy(dst_hbm.at[start + t], dsts[slot], isems[slot])

        idx_load(0, 0)
        idx_load(1, 1)

        def step4(g, carry):
            for b in range(ISLOTS):
                t = g * ISLOTS + b
                rb = b % NBUF

                @pl.when(t < nch)
                def _():
                    pltpu.make_async_copy(
                        src_hbm.at[start], srcs[b], isems[b]).wait()
                    pltpu.make_async_copy(
                        src_hbm.at[start], dsts[b], isems[b]).wait()

                    @pl.when(t >= NBUF)
                    def _():
                        pltpu.make_async_copy(
                            rows[rb], acc.at[dsts[b]], ssems[rb]).wait()

                    @pl.when(t + 2 < nch)
                    def _():
                        idx_load(t + 2, (b + 2) % ISLOTS)

                    pltpu.async_copy(
                        table_hbm.at[srcs[b]], rows[rb], gsems[rb]).wait()
                    pltpu.async_copy(
                        rows[rb], acc.at[dsts[b]], ssems[rb], add=True)
            return carry

        lax.fori_loop(0, -(-T_STEPS // ISLOTS), step4, 0)
        for rb in range(NBUF):
            pltpu.make_async_copy(rows[rb], acc.at[dsts[0]], ssems[rb]).wait()

        plsc.subcore_barrier()
        pltpu.sync_copy(acc.at[pl.ds(r0, ROWS_PER_TILE)],
                        out_hbm.at[cid, pl.ds(r0, ROWS_PER_TILE)])

    def deg_body(src_hbm, dst_hbm, out_hbm, dstall, ones, zbuf, acc, sem):
        cid, sid, start, nch = worker_range()

        fill(ones, CHUNK, 1.0)
        fill(zbuf, ZROWS, 0.0)
        pltpu.sync_copy(dst_hbm.at[pl.ds(start, T_STEPS)], dstall)
        r0 = sid * ROWS_PER_TILE
        for j in range(ROWS_PER_TILE // ZROWS):
            pltpu.sync_copy(zbuf, acc.at[pl.ds(r0 + j * ZROWS, ZROWS)])
        plsc.subcore_barrier()

        def step(t, carry):
            @pl.when(t < nch)
            def _():
                pltpu.async_copy(ones, acc.at[dstall.at[t]], sem, add=True)
            return carry

        lax.fori_loop(0, T_STEPS, step, 0)

        def drain(t, carry):
            @pl.when(t < nch)
            def _():
                pltpu.make_async_copy(ones, acc.at[dstall.at[t]], sem).wait()
            return carry

        lax.fori_loop(0, T_STEPS, drain, 0)
        plsc.subcore_barrier()
        pltpu.sync_copy(acc.at[pl.ds(r0, ROWS_PER_TILE)],
                        out_hbm.at[cid, pl.ds(r0, ROWS_PER_TILE)])

    return functools.partial(
        pl.kernel, out_type=out_type, mesh=mesh, scratch_types=scratch,
        compiler_params=pltpu.CompilerParams(use_tc_tiling_on_sc=False),
        name=f"sc_spmm_{d}_{int(gather)}")(gather_body if gather else deg_body)


_sc_deg = _make_sc_spmm(16, gather=False)
_sc_spmm128 = _make_sc_spmm(D_IN, gather=True)
_sc_spmm64 = _make_sc_spmm(D_OUT, gather=True)

_BLK = 1000
_GRID = N_NODES // _BLK


def _prescale_body(degp_ref, x_ref, dinv_ref, xs_ref):
    deg = degp_ref[0] + degp_ref[1] + 1.0
    dinv = lax.rsqrt(deg)[:, 0:1]
    dinv_ref[...] = dinv
    xs_ref[...] = dinv * x_ref[...]


_tc_prescale = pl.pallas_call(
    _prescale_body,
    grid=(_GRID,),
    in_specs=[
        pl.BlockSpec((NC, _BLK, 16), lambda i: (0, i, 0)),
        pl.BlockSpec((_BLK, D_IN), lambda i: (i, 0)),
    ],
    out_specs=[
        pl.BlockSpec((_BLK, 1), lambda i: (i, 0)),
        pl.BlockSpec((_BLK, D_IN), lambda i: (i, 0)),
    ],
    out_shape=[
        jax.ShapeDtypeStruct((N_NODES, 1), jnp.float32),
        jax.ShapeDtypeStruct((N_NODES, D_IN), jnp.float32),
    ],
)


def _mid_body(y1p_ref, xs_ref, dinv_ref, w1_ref, b1_ref, w2_ref, gs_ref):
    z = dinv_ref[...] * (y1p_ref[0] + y1p_ref[1] + xs_ref[...])
    h = jnp.dot(z, w1_ref[...], preferred_element_type=jnp.float32)
    h = jnp.maximum(h + b1_ref[...], 0.0)
    g = jnp.dot(h, w2_ref[...], preferred_element_type=jnp.float32)
    gs_ref[...] = dinv_ref[...] * g


_tc_mid = pl.pallas_call(
    _mid_body,
    grid=(_GRID,),
    in_specs=[
        pl.BlockSpec((NC, _BLK, D_IN), lambda i: (0, i, 0)),
        pl.BlockSpec((_BLK, D_IN), lambda i: (i, 0)),
        pl.BlockSpec((_BLK, 1), lambda i: (i, 0)),
        pl.BlockSpec((D_IN, D_HID), lambda i: (0, 0)),
        pl.BlockSpec((1, D_HID), lambda i: (0, 0)),
        pl.BlockSpec((D_HID, D_OUT), lambda i: (0, 0)),
    ],
    out_specs=pl.BlockSpec((_BLK, D_OUT), lambda i: (i, 0)),
    out_shape=jax.ShapeDtypeStruct((N_NODES, D_OUT), jnp.float32),
)


def _final_body(y2p_ref, gs_ref, dinv_ref, b2_ref, out_ref):
    t = dinv_ref[...] * (y2p_ref[0] + y2p_ref[1] + gs_ref[...]) + b2_ref[...]
    m = jnp.max(t, axis=1, keepdims=True)
    e = jnp.exp(t - m)
    s = jnp.sum(e, axis=1, keepdims=True)
    out_ref[...] = (t - m) - jnp.log(s)


_tc_final = pl.pallas_call(
    _final_body,
    grid=(_GRID,),
    in_specs=[
        pl.BlockSpec((NC, _BLK, D_OUT), lambda i: (0, i, 0)),
        pl.BlockSpec((_BLK, D_OUT), lambda i: (i, 0)),
        pl.BlockSpec((_BLK, 1), lambda i: (i, 0)),
        pl.BlockSpec((1, D_OUT), lambda i: (0, 0)),
    ],
    out_specs=pl.BlockSpec((_BLK, D_OUT), lambda i: (i, 0)),
    out_shape=jax.ShapeDtypeStruct((N_NODES, D_OUT), jnp.float32),
)


_CH_PAD = NC * NS * T_STEPS


@jax.jit
def kernel(x, edge_index, W1, b1, W2, b2):
    ei = jnp.pad(edge_index, ((0, 0), (0, (_CH_PAD - NCHUNKS) * CHUNK)))
    src = ei[0].reshape(_CH_PAD, CHUNK)
    dst = ei[1].reshape(_CH_PAD, CHUNK)
    degp = _sc_deg(src, dst)
    dinv, xs = _tc_prescale(degp, x)
    y1p = _sc_spmm128(xs, src, dst)
    gs = _tc_mid(y1p, xs, dinv, W1, b1.reshape(1, D_HID), W2)
    y2p = _sc_spmm64(gs, src, dst)
    return _tc_final(y2p, gs, dinv, b2.reshape(1, D_OUT))
